# Initial kernel scaffold; baseline (speedup 1.0000x reference)
import jax
import jax.numpy as jnp
from jax.experimental import pallas as pl


def kernel(x, edge_index, batch, W1a, b1a, W1b, b1b, g1, be1, W2a, b2a, W2b, b2b, g2, be2, Wf1, bf1, Wf2, bf2, Wl, bl):
    raise NotImplementedError("write your pallas kernel here")


# speedup vs baseline: 27.3679x; 27.3679x over previous
import functools

import jax
import jax.numpy as jnp
from jax import lax
from jax.experimental import pallas as pl
from jax.experimental.pallas import tpu as pltpu
from jax.experimental.pallas import tpu_sc as plsc

_N = 10000
_E = 320000
_F = 128
_DIM = 32
_C = 10
_G = 64

_NC = 2
_NS = 16
_NW = _NC * _NS
_EW = _E // _NW
_CH = 80
_CPW = _EW // _CH
_NP = 10240
_RPT = _NP // _NS
_YR = _N * _DIM // 128
_PR = _NC * _NP * _DIM // 128


_K = 5
_GRP = _CPW // _K


def _sc_segsum_body(y_hbm, src_hbm, dst_hbm, out_hbm,
                    src_v, dst_v, rows_a, rows_b, bounce_v, acc_s,
                    sem_ag, sem_bg, sem_as, sem_bs):
    c = lax.axis_index("c")
    s = lax.axis_index("s")
    wid = c * _NS + s

    pltpu.sync_copy(src_hbm.at[wid], src_v)
    pltpu.sync_copy(dst_hbm.at[wid], dst_v)

    def _fire(g, rows, semg):
        for b in range(_K):
            pltpu.async_copy(y_hbm.at[src_v.at[g * _K + b]], rows.at[b],
                             semg)

    def _consume(g, rows, semg, sems):
        cps = []
        for b in range(_K):
            pltpu.make_async_copy(y_hbm.at[src_v.at[g * _K + b]],
                                  rows.at[b], semg).wait()
            cps.append(pltpu.async_copy(rows.at[b],
                                        acc_s.at[dst_v.at[g * _K + b]],
                                        sems, add=True))
        for cp in cps:
            cp.wait()

    _fire(0, rows_a, sem_ag)

    def _zrow(i, carry):
        bounce_v[i, pl.ds(0, 16)] = jnp.zeros((16,), jnp.float32)
        bounce_v[i, pl.ds(16, 16)] = jnp.zeros((16,), jnp.float32)
        return carry
    lax.fori_loop(0, _RPT, _zrow, 0)
    pltpu.sync_copy(bounce_v, acc_s.at[pl.ds(s * _RPT, _RPT)])
    plsc.subcore_barrier()

    def _pair(j, carry):
        g0 = 2 * j
        _fire(g0 + 1, rows_b, sem_bg)
        _consume(g0, rows_a, sem_ag, sem_as)
        _fire(g0 + 2, rows_a, sem_ag)
        _consume(g0 + 1, rows_b, sem_bg, sem_bs)
        return carry
    lax.fori_loop(0, (_GRP - 1) // 2, _pair, 0)
    _consume(_GRP - 1, rows_a, sem_ag, sem_as)
    plsc.subcore_barrier()

    pltpu.sync_copy(acc_s.at[pl.ds(s * _RPT, _RPT)], bounce_v)
    pltpu.sync_copy(bounce_v, out_hbm.at[c, pl.ds(s * _RPT, _RPT)])


_sc_segsum = functools.partial(
    pl.kernel,
    out_type=jax.ShapeDtypeStruct((_NC, _NP, _DIM), jnp.float32),
    mesh=plsc.VectorSubcoreMesh(core_axis_name="c", subcore_axis_name="s"),
    compiler_params=pltpu.CompilerParams(use_tc_tiling_on_sc=False),
    scratch_types=[
        pltpu.VMEM((_CPW, _CH), jnp.int32),
        pltpu.VMEM((_CPW, _CH), jnp.int32),
        pltpu.VMEM((_K, _CH, _DIM), jnp.float32),
        pltpu.VMEM((_K, _CH, _DIM), jnp.float32),
        pltpu.VMEM((_RPT, _DIM), jnp.float32),
        pltpu.VMEM_SHARED((_NP, _DIM), jnp.float32),
        pltpu.SemaphoreType.DMA,
        pltpu.SemaphoreType.DMA,
        pltpu.SemaphoreType.DMA,
        pltpu.SemaphoreType.DMA,
    ],
)(_sc_segsum_body)


def _tc1_body(x_ref, w_ref, o_ref):
    o_ref[...] = jnp.dot(x_ref[...], w_ref[...],
                         preferred_element_type=jnp.float32)


def _bn_flat(h, g4, be4):
    def _fold(v):
        v32 = (v[:, :_DIM] + v[:, _DIM:2 * _DIM] + v[:, 2 * _DIM:3 * _DIM]
               + v[:, 3 * _DIM:]) * 0.25
        return jnp.concatenate([v32, v32, v32, v32], axis=1)
    mu = _fold(jnp.mean(h, axis=0, keepdims=True))
    d = h - mu
    var = _fold(jnp.mean(d * d, axis=0, keepdims=True))
    return d * lax.rsqrt(var + 1e-5) * g4 + be4


_P0 = _NP * _DIM // 128


def _tc2_body(y_ref, p_ref, b1a_ref, w1b_ref, b1b_ref, g1_ref, be1_ref,
              w2a_ref, o_ref):
    p = p_ref[...]
    t = jax.nn.relu(y_ref[...] + p[:_YR] + p[_P0:_P0 + _YR] + b1a_ref[...])
    h = jnp.dot(t, w1b_ref[...], preferred_element_type=jnp.float32)
    h = jax.nn.relu(h + b1b_ref[...])
    h = _bn_flat(h, g1_ref[...], be1_ref[...])
    o_ref[...] = jnp.dot(h, w2a_ref[...], preferred_element_type=jnp.float32)


def _tc3_body(m_ref, p_ref, b2a_ref, w2b_ref, b2b_ref, g2_ref, be2_ref,
              wf1_ref, bf1_ref, wf2_ref, bf2_ref, bat_ref, wl_ref, bl_ref,
              o_ref):
    p = p_ref[...]
    t = jax.nn.relu(m_ref[...] + p[:_YR] + p[_P0:_P0 + _YR] + b2a_ref[...])
    h = jnp.dot(t, w2b_ref[...], preferred_element_type=jnp.float32)
    h = jax.nn.relu(h + b2b_ref[...])
    h = _bn_flat(h, g2_ref[...], be2_ref[...])
    h3 = jax.nn.relu(jnp.dot(h, wf1_ref[...],
                             preferred_element_type=jnp.float32) + bf1_ref[...])
    h3 = jnp.dot(h3, wf2_ref[...],
                 preferred_element_type=jnp.float32) + bf2_ref[...]
    b4 = bat_ref[...]
    sums = jnp.zeros((_G, _DIM), jnp.float32)
    counts = jnp.zeros((_G, 1), jnp.float32)
    for j in range(4):
        ohj = (b4[j:j + 1] == lax.broadcasted_iota(jnp.int32, (_G, _YR), 0))
        ohj = ohj.astype(jnp.float32)
        sums = sums + jnp.dot(ohj, h3[:, _DIM * j:_DIM * (j + 1)],
                              preferred_element_type=jnp.float32)
        counts = counts + jnp.sum(ohj, axis=1, keepdims=True)
    pooled = sums / jnp.maximum(counts, 1.0)
    o_ref[...] = jnp.dot(pooled, wl_ref[...],
                         preferred_element_type=jnp.float32) + bl_ref[...]


def kernel(x, edge_index, batch, W1a, b1a, W1b, b1b, g1, be1, W2a, b2a,
           W2b, b2b, g2, be2, Wf1, bf1, Wf2, bf2, Wl, bl):
    src = edge_index[0].reshape(_NW, _CPW, _CH)
    dst = edge_index[1].reshape(_NW, _CPW, _CH)
    eye4 = jnp.eye(4, dtype=jnp.float32)

    def _t4(v):
        return jnp.tile(v, 4).reshape(1, 4 * _DIM)

    y1 = pl.pallas_call(
        _tc1_body,
        out_shape=jax.ShapeDtypeStruct((_YR, 128), jnp.float32),
    )(x.reshape(_YR, 4 * _F), jnp.kron(eye4, W1a))

    parts1 = _sc_segsum(y1.reshape(_N, _DIM), src, dst)

    m2 = pl.pallas_call(
        _tc2_body,
        out_shape=jax.ShapeDtypeStruct((_YR, 128), jnp.float32),
    )(y1, parts1.reshape(_PR, 128), _t4(b1a), jnp.kron(eye4, W1b),
      _t4(b1b), _t4(g1), _t4(be1), jnp.kron(eye4, W2a))

    parts2 = _sc_segsum(m2.reshape(_N, _DIM), src, dst)

    out = pl.pallas_call(
        _tc3_body,
        out_shape=jax.ShapeDtypeStruct((_G, _C), jnp.float32),
    )(m2, parts2.reshape(_PR, 128), _t4(b2a), jnp.kron(eye4, W2b),
      _t4(b2b), _t4(g2), _t4(be2), jnp.kron(eye4, Wf1),
      _t4(bf1), jnp.kron(eye4, Wf2), _t4(bf2),
      batch.reshape(_YR, 4).transpose(1, 0), Wl, bl.reshape(1, _C))
    return out
